# Initial kernel scaffold; baseline (speedup 1.0000x reference)
import jax
import jax.numpy as jnp
from jax.experimental import pallas as pl


def kernel(x, edge_index, W, att_src, att_dst, bias):
    raise NotImplementedError("write your pallas kernel here")


# speedup vs baseline: 27.4926x; 27.4926x over previous
import jax
import jax.numpy as jnp
from jax import lax
from jax.experimental import pallas as pl
from jax.experimental.pallas import tpu as pltpu
from jax.experimental.pallas import tpu_sc as plsc

N_NODES = 10000
CH = 128
LANES = 16
NP = 10240
N_DUMMY = NP - N_NODES
DEN_ROWS = NP // LANES
NCORES = 2
NSUB = 16
ROW_E = 64
EROWS = 5376
EP = EROWS * ROW_E
R1 = EROWS // NSUB
R2 = EROWS // (NCORES * NSUB)
B1 = 48
B2 = 24


def _leaky_exp(z):
    return jnp.exp(jnp.where(z >= 0, z, 0.2 * z))


def _sc_body(src_hbm, dst_hbm, asrc_hbm, adst_hbm, xp_hbm, outp_hbm,
             srcb, dstb, asrc_v, adst_v, den_v, alpha_v, rows_v, ridx_v,
             den_sh, out_sh):
    c = lax.axis_index("c")
    s = lax.axis_index("s")
    zero16 = jnp.zeros((LANES,), jnp.float32)
    iota16 = lax.iota(jnp.int32, LANES)

    pltpu.sync_copy(asrc_hbm, asrc_v)
    pltpu.sync_copy(adst_hbm, adst_v)

    def _zero_den(i, _):
        den_v[i] = zero16
        return 0
    lax.fori_loop(0, DEN_ROWS, _zero_den, 0)

    def _zero_rows(i, _):
        for k in range(CH // LANES):
            rows_v[i, pl.ds(k * LANES, LANES)] = zero16
        return 0
    lax.fori_loop(0, ROW_E, _zero_rows, 0)

    for j in range(5):
        for k in range(8):
            ridx_v[j, pl.ds(k * LANES, LANES)] = (j * 128 + k * LANES) + iota16

    pltpu.sync_copy(den_v.at[pl.ds(0, DEN_ROWS // NSUB)],
                    den_sh.at[pl.ds(s * (DEN_ROWS // NSUB), DEN_ROWS // NSUB)])
    for b in range(NP // NSUB // ROW_E):
        pltpu.sync_copy(
            rows_v, out_sh.at[pl.ds(s * (NP // NSUB) + b * ROW_E, ROW_E)])

    def _p1_block(bi, _):
        base = s * R1 + bi * B1
        pltpu.sync_copy(src_hbm.at[pl.ds(base, B1)], srcb)
        pltpu.sync_copy(dst_hbm.at[pl.ds(base, B1)], dstb)

        def _p1_row(j, _):
            for k in range(ROW_E // LANES):
                sv = srcb[j, pl.ds(k * LANES, LANES)]
                dv = dstb[j, pl.ds(k * LANES, LANES)]
                a1 = plsc.load_gather(asrc_v, [sv])
                a2 = plsc.load_gather(adst_v, [dv])
                ex = _leaky_exp(a1 + a2)
                plsc.addupdate_scatter(den_v, [dv >> 4, dv & 15], ex)
            return 0
        lax.fori_loop(0, B1, _p1_row, 0)
        return 0
    lax.fori_loop(0, R1 // B1, _p1_block, 0)

    plsc.subcore_barrier()
    for b in range(5):
        pltpu.sync_copy(den_v.at[pl.ds(b * 128, 128)],
                        den_sh.at[ridx_v.at[b]], add=True)
    plsc.subcore_barrier()
    pltpu.sync_copy(den_sh, den_v)

    def _p2_block(bi, _):
        base = c * (EROWS // NCORES) + s * R2 + bi * B2
        pltpu.sync_copy(src_hbm.at[pl.ds(base, B2)], srcb.at[pl.ds(0, B2)])
        pltpu.sync_copy(dst_hbm.at[pl.ds(base, B2)], dstb.at[pl.ds(0, B2)])

        def _chunk(j, _):
            pltpu.sync_copy(xp_hbm.at[srcb.at[j]], rows_v)
            for k in range(ROW_E // LANES):
                sv = srcb[j, pl.ds(k * LANES, LANES)]
                dv = dstb[j, pl.ds(k * LANES, LANES)]
                a1 = plsc.load_gather(asrc_v, [sv])
                a2 = plsc.load_gather(adst_v, [dv])
                ex = _leaky_exp(a1 + a2)
                den = plsc.load_gather(den_v, [dv >> 4, dv & 15])
                alpha_v[pl.ds(k * LANES, LANES)] = ex / den

            def _scale(g, _):
                av = alpha_v[pl.ds(g * LANES, LANES)]
                for i in range(LANES):
                    a = av[i]
                    r = g * LANES + i
                    for v in range(CH // LANES):
                        rows_v[r, pl.ds(v * LANES, LANES)] = (
                            rows_v[r, pl.ds(v * LANES, LANES)] * a)
                return 0
            lax.fori_loop(0, ROW_E // LANES, _scale, 0)

            pltpu.sync_copy(rows_v, out_sh.at[dstb.at[j]], add=True)
            return 0
        lax.fori_loop(0, B2, _chunk, 0)
        return 0
    lax.fori_loop(0, R2 // B2, _p2_block, 0)

    plsc.subcore_barrier()
    pltpu.sync_copy(out_sh.at[pl.ds(s * (NP // NSUB), NP // NSUB)],
                    outp_hbm.at[c, pl.ds(s * (NP // NSUB), NP // NSUB)])


def _mm_body(x_ref, w_ref, o_ref):
    o_ref[...] = jnp.dot(x_ref[...], w_ref[...],
                         preferred_element_type=jnp.float32)


def _fin_body(p_ref, b_ref, o_ref):
    o_ref[...] = p_ref[0] + p_ref[1] + b_ref[...]


@jax.jit
def kernel(x, edge_index, W, att_src, att_dst, bias):
    n = x.shape[0]
    e = edge_index.shape[1]
    att_s = att_src.reshape(CH)
    att_d = att_dst.reshape(CH)

    wext = jnp.concatenate(
        [W, (W @ att_s)[:, None], (W @ att_d)[:, None],
         jnp.zeros((CH, 126), jnp.float32)], axis=1)

    xe = pl.pallas_call(
        _mm_body,
        grid=(10,),
        in_specs=[pl.BlockSpec((1000, CH), lambda i: (i, 0)),
                  pl.BlockSpec((CH, 256), lambda i: (0, 0))],
        out_specs=pl.BlockSpec((1000, 256), lambda i: (i, 0)),
        out_shape=jax.ShapeDtypeStruct((n, 256), jnp.float32),
    )(x, wext)
    xp = xe[:, :CH]
    pad_sc = jnp.zeros((NP - n,), jnp.float32)
    asrc_p = jnp.concatenate([xe[:, CH], pad_sc])
    adst_p = jnp.concatenate([xe[:, CH + 1], pad_sc])

    loop = jnp.arange(n, dtype=edge_index.dtype)
    npad = EP - (e + n)
    pad_src = (jnp.arange(npad, dtype=jnp.int32) * 131) % n
    pad_dst = n + jnp.arange(npad, dtype=jnp.int32) % N_DUMMY
    src_p = jnp.concatenate([edge_index[0], loop, pad_src]).reshape(EROWS, ROW_E)
    dst_p = jnp.concatenate([edge_index[1], loop, pad_dst]).reshape(EROWS, ROW_E)

    mesh = plsc.VectorSubcoreMesh(core_axis_name="c", subcore_axis_name="s")
    outp = pl.kernel(
        _sc_body,
        out_type=jax.ShapeDtypeStruct((NCORES, NP, CH), jnp.float32),
        mesh=mesh,
        compiler_params=pltpu.CompilerParams(use_tc_tiling_on_sc=False,
                                             needs_layout_passes=False),
        scratch_types=[
            pltpu.VMEM((B1, ROW_E), jnp.int32),
            pltpu.VMEM((B1, ROW_E), jnp.int32),
            pltpu.VMEM((NP,), jnp.float32),
            pltpu.VMEM((NP,), jnp.float32),
            pltpu.VMEM((DEN_ROWS, LANES), jnp.float32),
            pltpu.VMEM((ROW_E,), jnp.float32),
            pltpu.VMEM((ROW_E, CH), jnp.float32),
            pltpu.VMEM((5, 128), jnp.int32),
            pltpu.VMEM_SHARED((DEN_ROWS, LANES), jnp.float32),
            pltpu.VMEM_SHARED((NP, CH), jnp.float32),
        ],
    )(src_p, dst_p, asrc_p, adst_p, xp)

    out = pl.pallas_call(
        _fin_body,
        grid=(10,),
        in_specs=[pl.BlockSpec((NCORES, 1000, CH), lambda i: (0, i, 0)),
                  pl.BlockSpec((1, CH), lambda i: (0, 0))],
        out_specs=pl.BlockSpec((1000, CH), lambda i: (i, 0)),
        out_shape=jax.ShapeDtypeStruct((n, CH), jnp.float32),
    )(outp, bias.reshape(1, CH))
    return out


# speedup vs baseline: 36.9409x; 1.3437x over previous
import jax
import jax.numpy as jnp
from jax import lax
from jax.experimental import pallas as pl
from jax.experimental.pallas import tpu as pltpu
from jax.experimental.pallas import tpu_sc as plsc

N_NODES = 10000
CH = 128
LANES = 16
NP = 10240
N_DUMMY = NP - N_NODES
DEN_ROWS = NP // LANES
NCORES = 2
NSUB = 16
ROW_E = 48
EROWS = 7168
EP = EROWS * ROW_E
R1 = EROWS // NSUB
R2 = EROWS // (NCORES * NSUB)
BLK = 32
NPAIR = BLK // 2


def _leaky_exp(z):
    return jnp.exp(jnp.where(z >= 0, z, 0.2 * z))


def _sc_body(src_hbm, dst_hbm, asrc_hbm, adst_hbm, xp_hbm, outp_hbm,
             srcb, dstb, asrc_v, adst_v, den_v, alpha_v, rows_a, rows_b,
             ridx_v, den_sh, out_sh, sem_ga, sem_gb, sem_sa, sem_sb):
    c = lax.axis_index("c")
    s = lax.axis_index("s")
    zero16 = jnp.zeros((LANES,), jnp.float32)
    iota16 = lax.iota(jnp.int32, LANES)

    pltpu.sync_copy(asrc_hbm, asrc_v)
    pltpu.sync_copy(adst_hbm, adst_v)

    def _zero_den(i, _):
        den_v[i] = zero16
        return 0
    lax.fori_loop(0, DEN_ROWS, _zero_den, 0)

    def _zero_rows(i, _):
        for k in range(CH // LANES):
            rows_a[i, pl.ds(k * LANES, LANES)] = zero16
            rows_b[i, pl.ds(k * LANES, LANES)] = zero16
        return 0
    lax.fori_loop(0, ROW_E, _zero_rows, 0)

    for j in range(5):
        for k in range(8):
            ridx_v[j, pl.ds(k * LANES, LANES)] = (j * 128 + k * LANES) + iota16

    pltpu.sync_copy(den_v.at[pl.ds(0, DEN_ROWS // NSUB)],
                    den_sh.at[pl.ds(s * (DEN_ROWS // NSUB), DEN_ROWS // NSUB)])
    obase = s * (NP // NSUB)
    for b in range(13):
        pltpu.sync_copy(rows_a, out_sh.at[pl.ds(obase + b * ROW_E, ROW_E)])
    pltpu.sync_copy(rows_a.at[pl.ds(0, 16)],
                    out_sh.at[pl.ds(obase + 13 * ROW_E, 16)])

    pltpu.async_copy(rows_b, out_sh.at[pl.ds(N_NODES, ROW_E)], sem_sb)

    def _p1_block(bi, _):
        base = s * R1 + bi * BLK
        pltpu.sync_copy(src_hbm.at[pl.ds(base, BLK)], srcb)
        pltpu.sync_copy(dst_hbm.at[pl.ds(base, BLK)], dstb)

        def _p1_row(j, _):
            for k in range(ROW_E // LANES):
                sv = srcb[j, pl.ds(k * LANES, LANES)]
                dv = dstb[j, pl.ds(k * LANES, LANES)]
                a1 = plsc.load_gather(asrc_v, [sv])
                a2 = plsc.load_gather(adst_v, [dv])
                ex = _leaky_exp(a1 + a2)
                plsc.addupdate_scatter(den_v, [dv >> 4, dv & 15], ex)
            return 0
        lax.fori_loop(0, BLK, _p1_row, 0)
        return 0
    lax.fori_loop(0, R1 // BLK, _p1_block, 0)

    plsc.subcore_barrier()
    for b in range(5):
        pltpu.sync_copy(den_v.at[pl.ds(b * 128, 128)],
                        den_sh.at[ridx_v.at[b]], add=True)
    plsc.subcore_barrier()
    pltpu.sync_copy(den_sh, den_v)

    def _alpha(j):
        for k in range(ROW_E // LANES):
            sv = srcb[j, pl.ds(k * LANES, LANES)]
            dv = dstb[j, pl.ds(k * LANES, LANES)]
            a1 = plsc.load_gather(asrc_v, [sv])
            a2 = plsc.load_gather(adst_v, [dv])
            ex = _leaky_exp(a1 + a2)
            den = plsc.load_gather(den_v, [dv >> 4, dv & 15])
            alpha_v[pl.ds(k * LANES, LANES)] = ex / den

    def _scale(rows):
        def body(g, _):
            av = alpha_v[pl.ds(g * LANES, LANES)]
            for i in range(LANES):
                a = av[i]
                r = g * LANES + i
                for v in range(CH // LANES):
                    rows[r, pl.ds(v * LANES, LANES)] = (
                        rows[r, pl.ds(v * LANES, LANES)] * a)
            return 0
        lax.fori_loop(0, ROW_E // LANES, body, 0)

    def _p2_block(bi, _):
        base = c * (EROWS // NCORES) + s * R2 + bi * BLK
        pltpu.sync_copy(src_hbm.at[pl.ds(base, BLK)], srcb)
        pltpu.sync_copy(dst_hbm.at[pl.ds(base, BLK)], dstb)
        pltpu.async_copy(xp_hbm.at[srcb.at[0]], rows_a, sem_ga)

        def _pair(p, _):
            j0 = 2 * p
            j1 = 2 * p + 1
            pltpu.make_async_copy(rows_b, out_sh.at[dstb.at[j1]], sem_sb).wait()
            pltpu.async_copy(xp_hbm.at[srcb.at[j1]], rows_b, sem_gb)
            _alpha(j0)
            pltpu.make_async_copy(xp_hbm.at[srcb.at[j0]], rows_a, sem_ga).wait()
            _scale(rows_a)
            pltpu.async_copy(rows_a, out_sh.at[dstb.at[j0]], sem_sa, add=True)
            _alpha(j1)
            pltpu.make_async_copy(xp_hbm.at[srcb.at[j1]], rows_b, sem_gb).wait()
            _scale(rows_b)
            pltpu.make_async_copy(rows_a, out_sh.at[dstb.at[j0]], sem_sa).wait()

            @pl.when(p < NPAIR - 1)
            def _():
                pltpu.async_copy(xp_hbm.at[srcb.at[j0 + 2]], rows_a, sem_ga)

            pltpu.async_copy(rows_b, out_sh.at[dstb.at[j1]], sem_sb, add=True)
            return 0
        lax.fori_loop(0, NPAIR, _pair, 0)
        return 0
    lax.fori_loop(0, R2 // BLK, _p2_block, 0)

    pltpu.make_async_copy(rows_b, out_sh.at[pl.ds(N_NODES, ROW_E)], sem_sb).wait()

    plsc.subcore_barrier()
    pltpu.sync_copy(out_sh.at[pl.ds(obase, NP // NSUB)],
                    outp_hbm.at[c, pl.ds(obase, NP // NSUB)])


def _mm_body(x_ref, w_ref, o_ref):
    o_ref[...] = jnp.dot(x_ref[...], w_ref[...],
                         preferred_element_type=jnp.float32)


def _fin_body(p_ref, b_ref, o_ref):
    o_ref[...] = p_ref[0] + p_ref[1] + b_ref[...]


@jax.jit
def kernel(x, edge_index, W, att_src, att_dst, bias):
    n = x.shape[0]
    e = edge_index.shape[1]
    att_s = att_src.reshape(CH)
    att_d = att_dst.reshape(CH)

    wext = jnp.concatenate(
        [W, (W @ att_s)[:, None], (W @ att_d)[:, None],
         jnp.zeros((CH, 126), jnp.float32)], axis=1)

    xe = pl.pallas_call(
        _mm_body,
        grid=(10,),
        in_specs=[pl.BlockSpec((1000, CH), lambda i: (i, 0)),
                  pl.BlockSpec((CH, 256), lambda i: (0, 0))],
        out_specs=pl.BlockSpec((1000, 256), lambda i: (i, 0)),
        out_shape=jax.ShapeDtypeStruct((n, 256), jnp.float32),
    )(x, wext)
    xp = xe[:, :CH]
    pad_sc = jnp.zeros((NP - n,), jnp.float32)
    asrc_p = jnp.concatenate([xe[:, CH], pad_sc])
    adst_p = jnp.concatenate([xe[:, CH + 1], pad_sc])

    loop = jnp.arange(n, dtype=edge_index.dtype)
    npad = EP - (e + n)
    pad_src = (jnp.arange(npad, dtype=jnp.int32) * 131) % n
    pad_dst = n + jnp.arange(npad, dtype=jnp.int32) % N_DUMMY
    src_p = jnp.concatenate([edge_index[0], loop, pad_src]).reshape(EROWS, ROW_E)
    dst_p = jnp.concatenate([edge_index[1], loop, pad_dst]).reshape(EROWS, ROW_E)

    mesh = plsc.VectorSubcoreMesh(core_axis_name="c", subcore_axis_name="s")
    outp = pl.kernel(
        _sc_body,
        out_type=jax.ShapeDtypeStruct((NCORES, NP, CH), jnp.float32),
        mesh=mesh,
        compiler_params=pltpu.CompilerParams(use_tc_tiling_on_sc=False,
                                             needs_layout_passes=False),
        scratch_types=[
            pltpu.VMEM((BLK, ROW_E), jnp.int32),
            pltpu.VMEM((BLK, ROW_E), jnp.int32),
            pltpu.VMEM((NP,), jnp.float32),
            pltpu.VMEM((NP,), jnp.float32),
            pltpu.VMEM((DEN_ROWS, LANES), jnp.float32),
            pltpu.VMEM((ROW_E,), jnp.float32),
            pltpu.VMEM((ROW_E, CH), jnp.float32),
            pltpu.VMEM((ROW_E, CH), jnp.float32),
            pltpu.VMEM((5, 128), jnp.int32),
            pltpu.VMEM_SHARED((DEN_ROWS, LANES), jnp.float32),
            pltpu.VMEM_SHARED((NP, CH), jnp.float32),
            pltpu.SemaphoreType.DMA,
            pltpu.SemaphoreType.DMA,
            pltpu.SemaphoreType.DMA,
            pltpu.SemaphoreType.DMA,
        ],
    )(src_p, dst_p, asrc_p, adst_p, xp)

    out = pl.pallas_call(
        _fin_body,
        grid=(10,),
        in_specs=[pl.BlockSpec((NCORES, 1000, CH), lambda i: (0, i, 0)),
                  pl.BlockSpec((1, CH), lambda i: (0, 0))],
        out_specs=pl.BlockSpec((1000, CH), lambda i: (i, 0)),
        out_shape=jax.ShapeDtypeStruct((n, CH), jnp.float32),
    )(outp, bias.reshape(1, CH))
    return out
